# baseline (speedup 1.0000x reference)
import jax
import jax.numpy as jnp
import numpy as np
from jax import lax
from jax.experimental import pallas as pl
from jax.experimental.pallas import tpu as pltpu
from jax.experimental.pallas import tpu_sc as plsc

D_MODEL = 128
VOCAB = 100000
B = 1024
L = 200

NC = 2
NS = 16
NW = NC * NS
ROWS = B * L
ROWS_PER_W = ROWS // NW
CHUNK = L
NCHUNK = ROWS_PER_W // CHUNK
LANES = 16
DSLICES = D_MODEL // LANES


def _pe_table() -> np.ndarray:
    pos = np.arange(L, dtype=np.float32)[:, None]
    dim = np.arange(0, D_MODEL, 2, dtype=np.float32)
    angle = pos / np.power(10000.0, dim / D_MODEL)
    pe = np.zeros((L, D_MODEL), dtype=np.float32)
    pe[:, 0::2] = np.sin(angle)
    pe[:, 1::2] = np.cos(angle)
    return pe


_PE = _pe_table()


def _sc_body(x_hbm, pe_hbm, table_hbm, out_hbm, idx_v, pe_v, bufs, gsems, osems):
    wid = lax.axis_index("s") * NC + lax.axis_index("c")
    base = wid * ROWS_PER_W

    pltpu.sync_copy(x_hbm.at[pl.ds(base, ROWS_PER_W)], idx_v)
    pltpu.sync_copy(pe_hbm, pe_v)

    def start_gather(c, slot):
        return pltpu.async_copy(
            table_hbm.at[idx_v.at[pl.ds(c * CHUNK, CHUNK)]], bufs[slot], gsems[slot]
        )

    start_gather(0, 0)

    @pl.loop(0, NCHUNK, step=2)
    def _group(c0):
        for b in range(2):
            c = c0 + b
            cur = b
            other = 1 - b

            pltpu.make_async_copy(
                table_hbm.at[idx_v.at[pl.ds(c * CHUNK, CHUNK)]],
                bufs[cur],
                gsems[cur],
            ).wait()

            @pl.when(c >= 1)
            def _():
                pltpu.make_async_copy(
                    bufs[other],
                    out_hbm.at[pl.ds(base + (c - 1) * CHUNK, CHUNK)],
                    osems[other],
                ).wait()

            @pl.loop(0, L, unroll=2)
            def _row(r):
                for s in range(DSLICES):
                    sl = pl.ds(s * LANES, LANES)
                    plsc.addupdate(bufs[cur].at[r, sl], pe_v[r, sl])

            pltpu.async_copy(
                bufs[cur],
                out_hbm.at[pl.ds(base + c * CHUNK, CHUNK)],
                osems[cur],
            )

            @pl.when(c + 1 < NCHUNK)
            def _():
                start_gather(c + 1, other)

    last = NCHUNK - 1
    pltpu.make_async_copy(
        bufs[last % 2],
        out_hbm.at[pl.ds(base + last * CHUNK, CHUNK)],
        osems[last % 2],
    ).wait()


@jax.jit
def _sc_embed(x_flat, pe, table):
    mesh = plsc.VectorSubcoreMesh(core_axis_name="c", subcore_axis_name="s")
    return pl.kernel(
        _sc_body,
        out_type=jax.ShapeDtypeStruct((ROWS, D_MODEL), jnp.float32),
        mesh=mesh,
        scratch_types=[
            pltpu.VMEM((ROWS_PER_W,), jnp.int32),
            pltpu.VMEM((L, D_MODEL), jnp.float32),
            [pltpu.VMEM((CHUNK, D_MODEL), jnp.float32) for _ in range(2)],
            [pltpu.SemaphoreType.DMA for _ in range(2)],
            [pltpu.SemaphoreType.DMA for _ in range(2)],
        ],
    )(x_flat, pe, table)


def kernel(x, table):
    x_flat = x.reshape(ROWS).astype(jnp.int32)
    pe = jnp.asarray(_PE)
    out = _sc_embed(x_flat, pe, table)
    return out.reshape(B, L, D_MODEL)


# speedup vs baseline: 1.2179x; 1.2179x over previous
import jax
import jax.numpy as jnp
import numpy as np
from jax import lax
from jax.experimental import pallas as pl
from jax.experimental.pallas import tpu as pltpu
from jax.experimental.pallas import tpu_sc as plsc

D_MODEL = 128
VOCAB = 100000
B = 1024
L = 200

NC = 2
NS = 16
NW = NC * NS
ROWS = B * L
ROWS_PER_W = ROWS // NW
SEQ_PER_CHUNK = 2
CHUNK = SEQ_PER_CHUNK * L
NCHUNK = ROWS_PER_W // CHUNK
LANES = 16
DSLICES = D_MODEL // LANES


def _pe_table() -> np.ndarray:
    pos = np.arange(L, dtype=np.float32)[:, None]
    dim = np.arange(0, D_MODEL, 2, dtype=np.float32)
    angle = pos / np.power(10000.0, dim / D_MODEL)
    pe = np.zeros((L, D_MODEL), dtype=np.float32)
    pe[:, 0::2] = np.sin(angle)
    pe[:, 1::2] = np.cos(angle)
    return pe


_PE = _pe_table()


def _sc_body(
    x_hbm, pe_hbm, table_hbm, out_hbm, idxs, pe_v, bufs, isems, gsems, osems
):
    wid = lax.axis_index("s") * NC + lax.axis_index("c")
    base = wid * ROWS_PER_W

    pltpu.sync_copy(pe_hbm, pe_v)

    def start_gather(slot):
        return pltpu.async_copy(
            table_hbm.at[idxs[slot]], bufs[slot], gsems[slot]
        )

    def idx_copy(c, slot):
        return pltpu.make_async_copy(
            x_hbm.at[pl.ds(base + c * CHUNK, CHUNK)], idxs[slot], isems[slot]
        )

    pltpu.sync_copy(x_hbm.at[pl.ds(base, CHUNK)], idxs[0])
    start_gather(0)
    idx_copy(1, 1).start()

    @pl.loop(0, NCHUNK, step=2)
    def _group(c0):
        for b in range(2):
            c = c0 + b
            cur = b
            oth = 1 - b

            pltpu.make_async_copy(
                table_hbm.at[idxs[cur]], bufs[cur], gsems[cur]
            ).wait()

            @pl.when(c >= 1)
            def _():
                pltpu.make_async_copy(
                    bufs[oth],
                    out_hbm.at[pl.ds(base + (c - 1) * CHUNK, CHUNK)],
                    osems[oth],
                ).wait()

            @pl.loop(0, L, unroll=2)
            def _row(r):
                for s in range(DSLICES):
                    sl = pl.ds(s * LANES, LANES)
                    p = pe_v[r, sl]
                    for q in range(SEQ_PER_CHUNK):
                        plsc.addupdate(bufs[cur].at[q * L + r, sl], p)

            pltpu.async_copy(
                bufs[cur],
                out_hbm.at[pl.ds(base + c * CHUNK, CHUNK)],
                osems[cur],
            )

            @pl.when(c + 1 < NCHUNK)
            def _():
                idx_copy(c + 1, oth).wait()
                start_gather(oth)

            @pl.when(c + 2 < NCHUNK)
            def _():
                idx_copy(c + 2, cur).start()

    last = NCHUNK - 1
    pltpu.make_async_copy(
        bufs[last % 2],
        out_hbm.at[pl.ds(base + last * CHUNK, CHUNK)],
        osems[last % 2],
    ).wait()


@jax.jit
def _sc_embed(x_flat, pe, table):
    mesh = plsc.VectorSubcoreMesh(core_axis_name="c", subcore_axis_name="s")
    return pl.kernel(
        _sc_body,
        out_type=jax.ShapeDtypeStruct((ROWS, D_MODEL), jnp.float32),
        mesh=mesh,
        scratch_types=[
            [pltpu.VMEM((CHUNK,), jnp.int32) for _ in range(2)],
            pltpu.VMEM((L, D_MODEL), jnp.float32),
            [pltpu.VMEM((CHUNK, D_MODEL), jnp.float32) for _ in range(2)],
            [pltpu.SemaphoreType.DMA for _ in range(2)],
            [pltpu.SemaphoreType.DMA for _ in range(2)],
            [pltpu.SemaphoreType.DMA for _ in range(2)],
        ],
    )(x_flat, pe, table)


def kernel(x, table):
    x_flat = x.reshape(ROWS).astype(jnp.int32)
    pe = jnp.asarray(_PE)
    out = _sc_embed(x_flat, pe, table)
    return out.reshape(B, L, D_MODEL)
